# Initial kernel scaffold; baseline (speedup 1.0000x reference)
import jax
import jax.numpy as jnp
from jax.experimental import pallas as pl


def kernel(user_emb, item_emb, adj_indices, adj_values):
    raise NotImplementedError("write your pallas kernel here")


# speedup vs baseline: 3.0717x; 3.0717x over previous
import functools

import jax
import jax.numpy as jnp
from jax import lax
from jax.experimental import pallas as pl
from jax.experimental.pallas import tpu as pltpu
from jax.experimental.pallas import tpu_sc as plsc

N_USERS = 30000
N_ITEMS = 20000
N = N_USERS + N_ITEMS
E = 800000
D = 64
H = 32
NC = 2
NS = 16
K = 128
CT = -(-E // (NS * K))
E_PAD = NS * K * CT
NP = 50048
NR = NP // NS
ZR = 136
NB = NR // ZR


def _sc_body(x0, cols, rows, vals, y1, y2, y3, mn, mie,
             acc, idx_c, idx_r, vbuf, rbuf, zbuf, mb0, mb1, mb2, mb3, sem):
    cid = lax.axis_index("c")
    sid = lax.axis_index("s")
    row0 = sid * NR
    base_col = cid * NP

    zero16 = jnp.zeros((16,), jnp.float32)
    for r in range(ZR):
        for h in range(0, H, 16):
            zbuf[r, pl.ds(h, 16)] = zero16

    def spmm(src, dst):
        for j in range(NB):
            pltpu.sync_copy(zbuf, acc.at[pl.ds(row0 + j * ZR, ZR)])
        plsc.subcore_barrier()

        ebase = sid * (CT * K)

        def chunk(c, carry):
            off = ebase + c * K
            pltpu.sync_copy(cols.at[pl.ds(off, K)], idx_c)
            pltpu.sync_copy(rows.at[pl.ds(off, K)], idx_r)
            pltpu.sync_copy(vals.at[pl.ds(off, K)], vbuf)
            for g in range(K // 16):
                sl = pl.ds(g * 16, 16)
                idx_c[sl] = idx_c[sl] + base_col
            pltpu.async_copy(src.at[idx_c], rbuf, sem).wait()
            def scale_group(g, carry):
                vv = vbuf[pl.ds(g * 16, 16)]
                for e in range(16):
                    i = g * 16 + e
                    v = vv[e]
                    for h in range(0, H, 16):
                        sl = pl.ds(h, 16)
                        rbuf[i, sl] = rbuf[i, sl] * v
                return carry

            lax.fori_loop(0, K // 16, scale_group, 0)
            pltpu.sync_copy(rbuf, acc.at[idx_r], add=True)
            return carry

        lax.fori_loop(0, CT, chunk, 0)
        plsc.subcore_barrier()

        for j in range(NB):
            r = row0 + j * ZR
            pltpu.sync_copy(acc.at[pl.ds(r, ZR)], dst.at[pl.ds(base_col + r, ZR)])
        plsc.subcore_barrier()

    spmm(x0, y1)
    spmm(y1, y2)
    spmm(y2, y3)

    def mean_block(j, carry):
        g0 = base_col + row0 + j * ZR
        sl_rows = pl.ds(g0, ZR)
        pltpu.sync_copy(x0.at[sl_rows], mb0)
        pltpu.sync_copy(y1.at[sl_rows], mb1)
        pltpu.sync_copy(y2.at[sl_rows], mb2)
        pltpu.sync_copy(y3.at[sl_rows], mb3)
        def mean_row(i, c2):
            for h in range(0, H, 16):
                sl = pl.ds(h, 16)
                mb0[i, sl] = (mb0[i, sl] + mb1[i, sl] + mb2[i, sl]
                              + mb3[i, sl]) * 0.25
            return c2

        lax.fori_loop(0, ZR, mean_row, 0)
        pltpu.sync_copy(mb0, mn.at[sl_rows])
        return carry

    lax.fori_loop(0, NB, mean_block, 0)
    plsc.subcore_barrier()

    spmm(mn, mie)


def kernel(user_emb, item_emb, adj_indices, adj_values):
    ego = jnp.concatenate([user_emb, item_emb], axis=0)
    rows = adj_indices[0].astype(jnp.int32)
    cols = adj_indices[1].astype(jnp.int32)
    vals = adj_values.astype(jnp.float32)

    pad = E_PAD - E
    pidx = (jnp.arange(pad, dtype=jnp.int32) * 61) % N
    rows_p = jnp.concatenate([rows, pidx])
    cols_p = jnp.concatenate([cols, pidx])
    vals_p = jnp.concatenate([vals, jnp.zeros((pad,), jnp.float32)])

    zpad = jnp.zeros((NP - N, H), jnp.float32)
    x0 = jnp.concatenate([ego[:, :H], zpad, ego[:, H:], zpad], axis=0)

    mesh = plsc.VectorSubcoreMesh(core_axis_name="c", subcore_axis_name="s")
    out_type = tuple(jax.ShapeDtypeStruct((2 * NP, H), jnp.float32)
                     for _ in range(5))
    scratch = [
        pltpu.VMEM_SHARED((NP, H), jnp.float32),
        pltpu.VMEM((K,), jnp.int32),
        pltpu.VMEM((K,), jnp.int32),
        pltpu.VMEM((K,), jnp.float32),
        pltpu.VMEM((K, H), jnp.float32),
        pltpu.VMEM((ZR, H), jnp.float32),
        pltpu.VMEM((ZR, H), jnp.float32),
        pltpu.VMEM((ZR, H), jnp.float32),
        pltpu.VMEM((ZR, H), jnp.float32),
        pltpu.VMEM((ZR, H), jnp.float32),
        pltpu.SemaphoreType.DMA,
    ]
    run = pl.kernel(_sc_body, out_type=out_type, mesh=mesh,
                    scratch_types=scratch,
                    compiler_params=pltpu.CompilerParams(
                        use_tc_tiling_on_sc=False))
    y1, y2, y3, mn, mie = run(x0, cols_p, rows_p, vals_p)

    def unsplit(t):
        return jnp.concatenate([t[:N], t[NP:NP + N]], axis=1)

    l1, l2, l3 = unsplit(y1), unsplit(y2), unsplit(y3)
    mean_emb = unsplit(mn)
    mean_item_emb = unsplit(mie)
    stacked = jnp.stack([ego, l1, l2, l3], axis=1)
    return (mean_emb[:N_USERS], mean_emb[N_USERS:], stacked, mean_item_emb)


# speedup vs baseline: 6.5178x; 2.1219x over previous
import functools

import jax
import jax.numpy as jnp
from jax import lax
from jax.experimental import pallas as pl
from jax.experimental.pallas import tpu as pltpu
from jax.experimental.pallas import tpu_sc as plsc

N_USERS = 30000
N_ITEMS = 20000
N = N_USERS + N_ITEMS
E = 800000
D = 64
H = 32
NC = 2
NS = 16
K = 128
SB = 8
CT = SB * (-(-E // (NS * K * SB)))
NSUP = CT // SB
E_PAD = NS * K * CT
NP = 50048
NR = NP // NS
ZR = 136
NB = NR // ZR


def _sc_body(x0, colsf, rows2, valsf, y1, y2, y3, mn, mie,
             acc, idxc, idxr, vbuf, rbuf0, rbuf1, zbuf, mb0, mb1, mb2,
             gsem0, gsem1, ssem0, ssem1, zsem):
    cid = lax.axis_index("c")
    sid = lax.axis_index("s")
    row0 = sid * NR
    base_col = cid * NP
    rbufs = (rbuf0, rbuf1)
    gsems = (gsem0, gsem1)
    ssems = (ssem0, ssem1)

    zero16 = jnp.zeros((16,), jnp.float32)
    for r in range(ZR):
        for h in range(0, H, 16):
            zbuf[r, pl.ds(h, 16)] = zero16

    def spmm(src, dst):
        zd = [pltpu.async_copy(zbuf, acc.at[pl.ds(row0 + j * ZR, ZR)], zsem)
              for j in range(NB)]
        for d in zd:
            d.wait()
        plsc.subcore_barrier()

        cbase = sid * CT

        def super_chunk(s, carry):
            c0 = cbase + s * SB
            pltpu.sync_copy(colsf.at[pl.ds(c0 * K, SB * K)], idxc)
            pltpu.sync_copy(rows2.at[pl.ds(c0, SB)], idxr)
            pltpu.sync_copy(valsf.at[pl.ds(c0 * K, SB * K)], vbuf)

            def add_off(g, c2):
                sl = pl.ds(g * 16, 16)
                idxc[sl] = idxc[sl] + base_col
                return c2

            lax.fori_loop(0, (SB * K) // 16, add_off, 0)

            gd = [None, None]
            sd = [None, None]
            gd[0] = pltpu.async_copy(src.at[idxc.at[pl.ds(0, K)]],
                                     rbufs[0], gsems[0])
            for j in range(SB):
                b = j & 1
                if j + 1 < SB:
                    if j >= 1:
                        sd[1 - b].wait()
                    gd[1 - b] = pltpu.async_copy(
                        src.at[idxc.at[pl.ds((j + 1) * K, K)]],
                        rbufs[1 - b], gsems[1 - b])
                gd[b].wait()

                def scale_group(g, c2, _j=j, _b=b):
                    vv = vbuf[pl.ds(_j * K + g * 16, 16)]
                    rb = rbufs[_b]
                    for e in range(16):
                        i = g * 16 + e
                        v = vv[e]
                        for h in range(0, H, 16):
                            sl = pl.ds(h, 16)
                            rb[i, sl] = rb[i, sl] * v
                    return c2

                lax.fori_loop(0, K // 16, scale_group, 0)
                sd[b] = pltpu.async_copy(rbufs[b], acc.at[idxr.at[j]],
                                         ssems[b], add=True)
            sd[0].wait()
            sd[1].wait()
            return carry

        lax.fori_loop(0, NSUP, super_chunk, 0)
        plsc.subcore_barrier()

        wd = [pltpu.async_copy(acc.at[pl.ds(row0 + j * ZR, ZR)],
                               dst.at[pl.ds(base_col + row0 + j * ZR, ZR)],
                               zsem)
              for j in range(NB)]
        for d in wd:
            d.wait()
        plsc.subcore_barrier()

    spmm(x0, y1)
    spmm(y1, y2)
    spmm(y2, y3)

    def mean_block(j, carry):
        g0 = base_col + row0 + j * ZR
        sl_rows = pl.ds(g0, ZR)
        pltpu.sync_copy(x0.at[sl_rows], mb0)
        pltpu.sync_copy(y1.at[sl_rows], mb1)
        pltpu.sync_copy(y2.at[sl_rows], mb2)
        pltpu.sync_copy(y3.at[sl_rows], zbuf)
        def mean_row(i, c2):
            for h in range(0, H, 16):
                sl = pl.ds(h, 16)
                mb0[i, sl] = (mb0[i, sl] + mb1[i, sl] + mb2[i, sl]
                              + zbuf[i, sl]) * 0.25
            return c2

        lax.fori_loop(0, ZR, mean_row, 0)
        pltpu.sync_copy(mb0, mn.at[sl_rows])
        return carry

    lax.fori_loop(0, NB, mean_block, 0)
    for r in range(ZR):
        for h in range(0, H, 16):
            zbuf[r, pl.ds(h, 16)] = zero16
    plsc.subcore_barrier()

    spmm(mn, mie)


def kernel(user_emb, item_emb, adj_indices, adj_values):
    ego = jnp.concatenate([user_emb, item_emb], axis=0)
    rows = adj_indices[0].astype(jnp.int32)
    cols = adj_indices[1].astype(jnp.int32)
    vals = adj_values.astype(jnp.float32)

    pad = E_PAD - E
    pidx = (jnp.arange(pad, dtype=jnp.int32) * 61) % N
    rows_p = jnp.concatenate([rows, pidx])
    cols_p = jnp.concatenate([cols, pidx])
    vals_p = jnp.concatenate([vals, jnp.zeros((pad,), jnp.float32)])

    zpad = jnp.zeros((NP - N, H), jnp.float32)
    x0 = jnp.concatenate([ego[:, :H], zpad, ego[:, H:], zpad], axis=0)

    mesh = plsc.VectorSubcoreMesh(core_axis_name="c", subcore_axis_name="s")
    out_type = tuple(jax.ShapeDtypeStruct((2 * NP, H), jnp.float32)
                     for _ in range(5))
    scratch = [
        pltpu.VMEM_SHARED((NP, H), jnp.float32),
        pltpu.VMEM((SB * K,), jnp.int32),
        pltpu.VMEM((SB, K), jnp.int32),
        pltpu.VMEM((SB * K,), jnp.float32),
        pltpu.VMEM((K, H), jnp.float32),
        pltpu.VMEM((K, H), jnp.float32),
        pltpu.VMEM((ZR, H), jnp.float32),
        pltpu.VMEM((ZR, H), jnp.float32),
        pltpu.VMEM((ZR, H), jnp.float32),
        pltpu.VMEM((ZR, H), jnp.float32),
        pltpu.SemaphoreType.DMA,
        pltpu.SemaphoreType.DMA,
        pltpu.SemaphoreType.DMA,
        pltpu.SemaphoreType.DMA,
        pltpu.SemaphoreType.DMA,
    ]
    run = pl.kernel(_sc_body, out_type=out_type, mesh=mesh,
                    scratch_types=scratch,
                    compiler_params=pltpu.CompilerParams(
                        use_tc_tiling_on_sc=False))
    y1, y2, y3, mn, mie = run(x0, cols_p, rows_p.reshape(E_PAD // K, K),
                              vals_p)

    def unsplit(t):
        return jnp.concatenate([t[:N], t[NP:NP + N]], axis=1)

    l1, l2, l3 = unsplit(y1), unsplit(y2), unsplit(y3)
    mean_emb = unsplit(mn)
    mean_item_emb = unsplit(mie)
    stacked = jnp.stack([ego, l1, l2, l3], axis=1)
    return (mean_emb[:N_USERS], mean_emb[N_USERS:], stacked, mean_item_emb)


# speedup vs baseline: 6.9547x; 1.0670x over previous
import functools

import jax
import jax.numpy as jnp
from jax import lax
from jax.experimental import pallas as pl
from jax.experimental.pallas import tpu as pltpu
from jax.experimental.pallas import tpu_sc as plsc

N_USERS = 30000
N_ITEMS = 20000
N = N_USERS + N_ITEMS
E = 800000
D = 64
H = 32
NC = 2
NS = 16
K = 128
SB = 8
CT = SB * (-(-E // (NS * K * SB)))
NSUP = CT // SB
E_PAD = NS * K * CT
NP = 50048
NR = NP // NS
ZR = 136
NB = NR // ZR


def _sc_body(x0, colsf, rows2, valsf, y1, y2, y3, mn, mie,
             acc, idxc, idxr, vbuf, gbuf0, gbuf1, sbuf0, sbuf1, zbuf,
             gsem0, gsem1, ssem0, ssem1, zsem):
    cid = lax.axis_index("c")
    sid = lax.axis_index("s")
    row0 = sid * NR
    base_col = cid * NP
    gbufs = (gbuf0, gbuf1)
    sbufs = (sbuf0, sbuf1)
    gsems = (gsem0, gsem1)
    ssems = (ssem0, ssem1)

    zero16 = jnp.zeros((16,), jnp.float32)
    for r in range(ZR):
        for h in range(0, H, 16):
            zbuf[r, pl.ds(h, 16)] = zero16

    def spmm(src, dst):
        zd = [pltpu.async_copy(zbuf, acc.at[pl.ds(row0 + j * ZR, ZR)], zsem)
              for j in range(NB)]
        for d in zd:
            d.wait()
        plsc.subcore_barrier()

        cbase = sid * CT

        def super_chunk(s, carry):
            c0 = cbase + s * SB
            pltpu.sync_copy(colsf.at[pl.ds(c0 * K, SB * K)], idxc)
            pltpu.sync_copy(rows2.at[pl.ds(c0, SB)], idxr)
            pltpu.sync_copy(valsf.at[pl.ds(c0 * K, SB * K)], vbuf)

            def add_off(g, c2):
                sl = pl.ds(g * 16, 16)
                idxc[sl] = idxc[sl] + base_col
                return c2

            lax.fori_loop(0, (SB * K) // 16, add_off, 0)

            gd = [None, None]
            sd = [None, None]
            gd[0] = pltpu.async_copy(src.at[idxc.at[pl.ds(0, K)]],
                                     gbufs[0], gsems[0])
            for j in range(SB):
                b = j & 1
                if j + 1 < SB:
                    gd[1 - b] = pltpu.async_copy(
                        src.at[idxc.at[pl.ds((j + 1) * K, K)]],
                        gbufs[1 - b], gsems[1 - b])
                gd[b].wait()
                if j >= 2:
                    sd[b].wait()

                def scale_group(g, c2, _j=j, _b=b):
                    vv = vbuf[pl.ds(_j * K + g * 16, 16)]
                    gb = gbufs[_b]
                    sb = sbufs[_b]
                    for e in range(16):
                        i = g * 16 + e
                        v = vv[e]
                        for h in range(0, H, 16):
                            sl = pl.ds(h, 16)
                            sb[i, sl] = gb[i, sl] * v
                    return c2

                lax.fori_loop(0, K // 16, scale_group, 0)
                sd[b] = pltpu.async_copy(sbufs[b], acc.at[idxr.at[j]],
                                         ssems[b], add=True)
            sd[0].wait()
            sd[1].wait()
            return carry

        lax.fori_loop(0, NSUP, super_chunk, 0)
        plsc.subcore_barrier()

        wd = [pltpu.async_copy(acc.at[pl.ds(row0 + j * ZR, ZR)],
                               dst.at[pl.ds(base_col + row0 + j * ZR, ZR)],
                               zsem)
              for j in range(NB)]
        for d in wd:
            d.wait()
        plsc.subcore_barrier()

    spmm(x0, y1)
    spmm(y1, y2)
    spmm(y2, y3)

    def mean_rows(nrows, goff):
        sl_rows = pl.ds(goff, nrows)
        pltpu.sync_copy(x0.at[sl_rows], gbuf0.at[pl.ds(0, nrows)])
        pltpu.sync_copy(y1.at[sl_rows], gbuf1.at[pl.ds(0, nrows)])
        pltpu.sync_copy(y2.at[sl_rows], sbuf0.at[pl.ds(0, nrows)])
        pltpu.sync_copy(y3.at[sl_rows], sbuf1.at[pl.ds(0, nrows)])

        def mean_row(i, c2):
            for h in range(0, H, 16):
                sl = pl.ds(h, 16)
                gbuf0[i, sl] = (gbuf0[i, sl] + gbuf1[i, sl] + sbuf0[i, sl]
                                + sbuf1[i, sl]) * 0.25
            return c2

        lax.fori_loop(0, nrows, mean_row, 0)
        pltpu.sync_copy(gbuf0.at[pl.ds(0, nrows)], mn.at[sl_rows])

    MR = K
    NMB = NR // MR
    MT = NR - NMB * MR

    def mean_block(j, carry):
        mean_rows(MR, base_col + row0 + j * MR)
        return carry

    lax.fori_loop(0, NMB, mean_block, 0)
    mean_rows(MT, base_col + row0 + NMB * MR)
    plsc.subcore_barrier()

    spmm(mn, mie)


def kernel(user_emb, item_emb, adj_indices, adj_values):
    ego = jnp.concatenate([user_emb, item_emb], axis=0)
    rows = adj_indices[0].astype(jnp.int32)
    cols = adj_indices[1].astype(jnp.int32)
    vals = adj_values.astype(jnp.float32)

    pad = E_PAD - E
    pidx = (jnp.arange(pad, dtype=jnp.int32) * 61) % N
    rows_p = jnp.concatenate([rows, pidx])
    cols_p = jnp.concatenate([cols, pidx])
    vals_p = jnp.concatenate([vals, jnp.zeros((pad,), jnp.float32)])

    zpad = jnp.zeros((NP - N, H), jnp.float32)
    x0 = jnp.concatenate([ego[:, :H], zpad, ego[:, H:], zpad], axis=0)

    mesh = plsc.VectorSubcoreMesh(core_axis_name="c", subcore_axis_name="s")
    out_type = tuple(jax.ShapeDtypeStruct((2 * NP, H), jnp.float32)
                     for _ in range(5))
    scratch = [
        pltpu.VMEM_SHARED((NP, H), jnp.float32),
        pltpu.VMEM((SB * K,), jnp.int32),
        pltpu.VMEM((SB, K), jnp.int32),
        pltpu.VMEM((SB * K,), jnp.float32),
        pltpu.VMEM((K, H), jnp.float32),
        pltpu.VMEM((K, H), jnp.float32),
        pltpu.VMEM((K, H), jnp.float32),
        pltpu.VMEM((K, H), jnp.float32),
        pltpu.VMEM((ZR, H), jnp.float32),
        pltpu.SemaphoreType.DMA,
        pltpu.SemaphoreType.DMA,
        pltpu.SemaphoreType.DMA,
        pltpu.SemaphoreType.DMA,
        pltpu.SemaphoreType.DMA,
    ]
    run = pl.kernel(_sc_body, out_type=out_type, mesh=mesh,
                    scratch_types=scratch,
                    compiler_params=pltpu.CompilerParams(
                        use_tc_tiling_on_sc=False))
    y1, y2, y3, mn, mie = run(x0, cols_p, rows_p.reshape(E_PAD // K, K),
                              vals_p)

    def unsplit(t):
        return jnp.concatenate([t[:N], t[NP:NP + N]], axis=1)

    l1, l2, l3 = unsplit(y1), unsplit(y2), unsplit(y3)
    mean_emb = unsplit(mn)
    mean_item_emb = unsplit(mie)
    stacked = jnp.stack([ego, l1, l2, l3], axis=1)
    return (mean_emb[:N_USERS], mean_emb[N_USERS:], stacked, mean_item_emb)


# speedup vs baseline: 8.2460x; 1.1857x over previous
import functools

import jax
import jax.numpy as jnp
from jax import lax
from jax.experimental import pallas as pl
from jax.experimental.pallas import tpu as pltpu
from jax.experimental.pallas import tpu_sc as plsc

N_USERS = 30000
N_ITEMS = 20000
N = N_USERS + N_ITEMS
E = 800000
D = 64
H = 32
NC = 2
NS = 16
K = 128
SB = 8
CT = SB * (-(-E // (NS * K * SB)))
NSUP = CT // SB
E_PAD = NS * K * CT
NP = 50048
NR = NP // NS
ZR = 136
NB = NR // ZR


def _sc_body(x0, colsb, rows2, valsf, y1, y2, y3, mn, mie,
             acc, idxc, idxr, vbuf, gbuf0, gbuf1, sbuf0, sbuf1, zbuf,
             gsem0, gsem1, ssem0, ssem1, zsem, isem):
    cid = lax.axis_index("c")
    sid = lax.axis_index("s")
    row0 = sid * NR
    base_col = cid * NP
    gbufs = (gbuf0, gbuf1)
    sbufs = (sbuf0, sbuf1)
    gsems = (gsem0, gsem1)
    ssems = (ssem0, ssem1)
    cbase = sid * CT

    zero16 = jnp.zeros((16,), jnp.float32)
    for r in range(ZR):
        for h in range(0, H, 16):
            zbuf[r, pl.ds(h, 16)] = zero16

    def fetch_super(s, p):
        c0 = cbase + s * SB
        pltpu.async_copy(colsb.at[cid, pl.ds(c0 * K, SB * K)],
                         idxc.at[p], isem)
        pltpu.async_copy(rows2.at[pl.ds(c0, SB)], idxr.at[p], isem)
        pltpu.async_copy(valsf.at[pl.ds(c0 * K, SB * K)], vbuf.at[p], isem)

    def drain_super():
        pltpu.make_async_copy(colsb.at[0, pl.ds(0, SB * K)], idxc.at[0],
                              isem).wait()
        pltpu.make_async_copy(rows2.at[pl.ds(0, SB)], idxr.at[0],
                              isem).wait()
        pltpu.make_async_copy(valsf.at[pl.ds(0, SB * K)], vbuf.at[0],
                              isem).wait()

    def spmm(src, dst):
        fetch_super(0, 0)
        zd = [pltpu.async_copy(zbuf, acc.at[pl.ds(row0 + j * ZR, ZR)], zsem)
              for j in range(NB)]
        for d in zd:
            d.wait()
        plsc.subcore_barrier()

        def super_chunk(s, carry):
            p = s & 1
            drain_super()

            @pl.when(s + 1 < NSUP)
            def _():
                fetch_super(s + 1, 1 - p)

            gd = [None, None]
            sd = [None, None]
            gd[0] = pltpu.async_copy(src.at[idxc.at[p, pl.ds(0, K)]],
                                     gbufs[0], gsems[0])
            for j in range(SB):
                b = j & 1
                if j + 1 < SB:
                    gd[1 - b] = pltpu.async_copy(
                        src.at[idxc.at[p, pl.ds((j + 1) * K, K)]],
                        gbufs[1 - b], gsems[1 - b])
                gd[b].wait()
                if j >= 2:
                    sd[b].wait()

                def scale_group(g, c2, _j=j, _b=b):
                    vv = vbuf[p, pl.ds(_j * K + g * 16, 16)]
                    gb = gbufs[_b]
                    sb = sbufs[_b]
                    for e in range(16):
                        i = g * 16 + e
                        v = vv[e]
                        for h in range(0, H, 16):
                            sl = pl.ds(h, 16)
                            sb[i, sl] = gb[i, sl] * v
                    return c2

                lax.fori_loop(0, K // 16, scale_group, 0)
                sd[b] = pltpu.async_copy(sbufs[b], acc.at[idxr.at[p, j]],
                                         ssems[b], add=True)
            sd[0].wait()
            sd[1].wait()
            return carry

        lax.fori_loop(0, NSUP, super_chunk, 0)
        plsc.subcore_barrier()

        wd = [pltpu.async_copy(acc.at[pl.ds(row0 + j * ZR, ZR)],
                               dst.at[pl.ds(base_col + row0 + j * ZR, ZR)],
                               zsem)
              for j in range(NB)]
        for d in wd:
            d.wait()
        plsc.subcore_barrier()

    spmm(x0, y1)
    spmm(y1, y2)
    spmm(y2, y3)

    def mean_rows(nrows, goff):
        sl_rows = pl.ds(goff, nrows)
        pltpu.sync_copy(x0.at[sl_rows], gbuf0.at[pl.ds(0, nrows)])
        pltpu.sync_copy(y1.at[sl_rows], gbuf1.at[pl.ds(0, nrows)])
        pltpu.sync_copy(y2.at[sl_rows], sbuf0.at[pl.ds(0, nrows)])
        pltpu.sync_copy(y3.at[sl_rows], sbuf1.at[pl.ds(0, nrows)])

        def mean_row(i, c2):
            for h in range(0, H, 16):
                sl = pl.ds(h, 16)
                gbuf0[i, sl] = (gbuf0[i, sl] + gbuf1[i, sl] + sbuf0[i, sl]
                                + sbuf1[i, sl]) * 0.25
            return c2

        lax.fori_loop(0, nrows, mean_row, 0)
        pltpu.sync_copy(gbuf0.at[pl.ds(0, nrows)], mn.at[sl_rows])

    MR = K
    NMB = NR // MR
    MT = NR - NMB * MR

    def mean_block(j, carry):
        mean_rows(MR, base_col + row0 + j * MR)
        return carry

    lax.fori_loop(0, NMB, mean_block, 0)
    mean_rows(MT, base_col + row0 + NMB * MR)
    plsc.subcore_barrier()

    spmm(mn, mie)


def kernel(user_emb, item_emb, adj_indices, adj_values):
    ego = jnp.concatenate([user_emb, item_emb], axis=0)
    rows = adj_indices[0].astype(jnp.int32)
    cols = adj_indices[1].astype(jnp.int32)
    vals = adj_values.astype(jnp.float32)

    pad = E_PAD - E
    pidx = (jnp.arange(pad, dtype=jnp.int32) * 61) % N
    rows_p = jnp.concatenate([rows, pidx])
    cols_p = jnp.concatenate([cols, pidx])
    vals_p = jnp.concatenate([vals, jnp.zeros((pad,), jnp.float32)])

    zpad = jnp.zeros((NP - N, H), jnp.float32)
    x0 = jnp.concatenate([ego[:, :H], zpad, ego[:, H:], zpad], axis=0)

    mesh = plsc.VectorSubcoreMesh(core_axis_name="c", subcore_axis_name="s")
    out_type = tuple(jax.ShapeDtypeStruct((2 * NP, H), jnp.float32)
                     for _ in range(5))
    scratch = [
        pltpu.VMEM_SHARED((NP, H), jnp.float32),
        pltpu.VMEM((2, SB * K), jnp.int32),
        pltpu.VMEM((2, SB, K), jnp.int32),
        pltpu.VMEM((2, SB * K), jnp.float32),
        pltpu.VMEM((K, H), jnp.float32),
        pltpu.VMEM((K, H), jnp.float32),
        pltpu.VMEM((K, H), jnp.float32),
        pltpu.VMEM((K, H), jnp.float32),
        pltpu.VMEM((ZR, H), jnp.float32),
        pltpu.SemaphoreType.DMA,
        pltpu.SemaphoreType.DMA,
        pltpu.SemaphoreType.DMA,
        pltpu.SemaphoreType.DMA,
        pltpu.SemaphoreType.DMA,
        pltpu.SemaphoreType.DMA,
    ]
    run = pl.kernel(_sc_body, out_type=out_type, mesh=mesh,
                    scratch_types=scratch,
                    compiler_params=pltpu.CompilerParams(
                        use_tc_tiling_on_sc=False))
    colsb = jnp.stack([cols_p, cols_p + NP])
    y1, y2, y3, mn, mie = run(x0, colsb, rows_p.reshape(E_PAD // K, K),
                              vals_p)

    def unsplit(t):
        return jnp.concatenate([t[:N], t[NP:NP + N]], axis=1)

    l1, l2, l3 = unsplit(y1), unsplit(y2), unsplit(y3)
    mean_emb = unsplit(mn)
    mean_item_emb = unsplit(mie)
    stacked = jnp.stack([ego, l1, l2, l3], axis=1)
    return (mean_emb[:N_USERS], mean_emb[N_USERS:], stacked, mean_item_emb)
